# Initial kernel scaffold; baseline (speedup 1.0000x reference)
import jax
import jax.numpy as jnp
from jax.experimental import pallas as pl


def kernel(xyz, features, W0_0, b0_0, W0_1, b0_1, W1_0, b1_0, W1_1, b1_1, W2_0, b2_0, W2_1, b2_1):
    raise NotImplementedError("write your pallas kernel here")


# speedup vs baseline: 19.9086x; 19.9086x over previous
import functools

import jax
import jax.numpy as jnp
from jax import lax
from jax.experimental import pallas as pl
from jax.experimental.pallas import tpu as pltpu
from jax.experimental.pallas import tpu_sc as plsc

B, N, C, D = 4, 4096, 3, 64
S = 1024
RADII2 = (0.2 * 0.2, 0.4 * 0.4, 0.8 * 0.8)
KS = (16, 32, 64)
SEG = (0, 16, 48)
TOTK = 112
INV_BN = 1.0 / (1.0 + 1e-5) ** 0.5
NCHUNK = N // 16


def _fps_body(x_ref, zi_ref, zf_ref, idx_ref):
    X = x_ref[0]
    Y = x_ref[1]
    Z = x_ref[2]
    jiota = lax.broadcasted_iota(jnp.int32, (B, N), 1)
    liota = lax.broadcasted_iota(jnp.int32, (B, S), 1)

    def body(i, state):
        dist, far, cent = state
        cent = jnp.where(liota == i, far, cent)
        fmask = jiota == far
        neg = jnp.float32(-3e38)
        cx = jnp.max(jnp.where(fmask, X, neg), axis=1, keepdims=True)
        cy = jnp.max(jnp.where(fmask, Y, neg), axis=1, keepdims=True)
        cz = jnp.max(jnp.where(fmask, Z, neg), axis=1, keepdims=True)
        dx = X - cx
        dy = Y - cy
        dz = Z - cz
        d = (dx * dx + dy * dy) + dz * dz
        dist = jnp.minimum(dist, d)
        m = jnp.max(dist, axis=1, keepdims=True)
        far = jnp.min(jnp.where(dist == m, jiota, jnp.int32(2**30)),
                      axis=1, keepdims=True)
        return dist, far, cent

    dist0 = zf_ref[...] + jnp.float32(1e10)
    far0 = zi_ref[:, 0:1]
    cent0 = zi_ref[...]
    _, _, cent = lax.fori_loop(0, S, body, (dist0, far0, cent0))
    idx_ref[...] = cent


def _fps(xt):
    zi = jnp.zeros((B, S), jnp.int32)
    zf = jnp.zeros((B, N), jnp.float32)
    return pl.pallas_call(
        _fps_body,
        out_shape=jax.ShapeDtypeStruct((B, S), jnp.int32),
    )(xt, zi, zf)


_MLP_BLK = 512


def _mlp_body(f_ref, w00, b00, w01, b01, w10, b10, w11, b11, w20, b20, w21,
              b21, y0_ref, y1_ref, y2_ref):
    f = f_ref[...]
    inv = jnp.float32(INV_BN)

    def layer(x, w_ref, b_ref):
        h = lax.dot_general(x, w_ref[...], (((1,), (1,)), ((), ())),
                            preferred_element_type=jnp.float32,
                            precision=lax.Precision.HIGHEST)
        return jnp.maximum((h + b_ref[...][None, :]) * inv, 0.0)

    y0_ref[...] = layer(layer(f, w00, b00), w01, b01)
    y1_ref[...] = layer(layer(f, w10, b10), w11, b11)
    y2_ref[...] = layer(layer(f, w20, b20), w21, b21)


def _mlp(feats, weights):
    grid = (B * N // _MLP_BLK,)
    fspec = pl.BlockSpec((_MLP_BLK, D), lambda i: (i, 0))
    wspecs = []
    for w in weights:
        nd = len(w.shape)
        wspecs.append(pl.BlockSpec(w.shape, (lambda i: (0, 0)) if nd == 2
                                   else (lambda i: (0,))))
    outspec = [
        pl.BlockSpec((_MLP_BLK, 128), lambda i: (i, 0)),
        pl.BlockSpec((_MLP_BLK, 128), lambda i: (i, 0)),
        pl.BlockSpec((_MLP_BLK, 128), lambda i: (i, 0)),
    ]
    outshape = [jax.ShapeDtypeStruct((B * N, 128), jnp.float32)] * 3
    return pl.pallas_call(
        _mlp_body,
        grid=grid,
        in_specs=[fspec] + wspecs,
        out_specs=outspec,
        out_shape=outshape,
    )(feats, *weights)


_SBLK = S * B // 32


def _sc_body(xf_hbm, yf_hbm, zf_hbm, fps_hbm, y0_hbm, y1_hbm, y2_hbm,
             nx_hbm, pts_hbm,
             x_v, y_v, z_v, fidx_v, cxs_v, cys_v, czs_v, idx_v, rows_v,
             blk_v, sem):
    ncores = 2
    wid = lax.axis_index("s") * ncores + lax.axis_index("c")
    b = wid // 8
    s0 = (wid % 8) * _SBLK

    pltpu.sync_copy(xf_hbm.at[pl.ds(b * N, N)], x_v)
    pltpu.sync_copy(yf_hbm.at[pl.ds(b * N, N)], y_v)
    pltpu.sync_copy(zf_hbm.at[pl.ds(b * N, N)], z_v)
    pltpu.sync_copy(fps_hbm.at[pl.ds(b * S + s0, _SBLK)], fidx_v)

    iota = lax.iota(jnp.int32, 16)

    def cgather(t, _):
        iv = fidx_v[pl.ds(t * 16, 16)]
        cxs_v[pl.ds(t * 16, 16)] = plsc.load_gather(x_v, [iv])
        cys_v[pl.ds(t * 16, 16)] = plsc.load_gather(y_v, [iv])
        czs_v[pl.ds(t * 16, 16)] = plsc.load_gather(z_v, [iv])
        return 0

    lax.fori_loop(0, _SBLK // 16, cgather, 0, unroll=True)
    pltpu.sync_copy(cxs_v, nx_hbm.at[pl.ds((b * 3 + 0) * S + s0, _SBLK)])
    pltpu.sync_copy(cys_v, nx_hbm.at[pl.ds((b * 3 + 1) * S + s0, _SBLK)])
    pltpu.sync_copy(czs_v, nx_hbm.at[pl.ds((b * 3 + 2) * S + s0, _SBLK)])

    base = b * N

    def per_centroid(c, _):
        cfull = jnp.full((16,), c, jnp.int32)
        cxv = plsc.load_gather(cxs_v, [cfull])
        cyv = plsc.load_gather(cys_v, [cfull])
        czv = plsc.load_gather(czs_v, [cfull])

        def cond(state):
            q, c0, c1, c2 = state
            return (q < NCHUNK) & ((c0 < KS[0]) | (c1 < KS[1]) | (c2 < KS[2]))

        def chunk(state):
            q, c0, c1, c2 = state
            px = x_v[pl.ds(q * 16, 16)]
            py = y_v[pl.ds(q * 16, 16)]
            pz = z_v[pl.ds(q * 16, 16)]
            dx = px - cxv
            dy = py - cyv
            dz = pz - czv
            d2 = (dx * dx + dy * dy) + dz * dz
            jv = iota + (q * 16 + base)
            cnts = [c0, c1, c2]
            new = []
            for bi in range(3):
                cnt = cnts[bi]

                def take(cnt=cnt, bi=bi):
                    m = d2 <= RADII2[bi]
                    rank = plsc.cumsum(jnp.where(m, 1, 0))
                    sel = m & (rank <= (KS[bi] - cnt))
                    plsc.store_compressed(idx_v.at[pl.ds(SEG[bi] + cnt, 16)],
                                          jv, mask=sel)
                    npc = plsc.all_reduce_population_count(sel)
                    return cnt + jnp.max(npc)

                new.append(lax.cond(cnt < KS[bi], take, lambda cnt=cnt: cnt))
            return q + 1, new[0], new[1], new[2]

        q, c0, c1, c2 = lax.while_loop(
            cond, chunk, (jnp.int32(0), jnp.int32(0), jnp.int32(0),
                          jnp.int32(0)))

        cnts = [c0, c1, c2]
        for bi in range(3):
            firstv = plsc.load_gather(
                idx_v, [jnp.full((16,), SEG[bi], jnp.int32)])

            def pcond(cnt):
                return cnt < KS[bi]

            def pbody(cnt, bi=bi, firstv=firstv):
                rem = KS[bi] - cnt
                m = iota < rem
                plsc.store_compressed(idx_v.at[pl.ds(SEG[bi] + cnt, 16)],
                                      firstv, mask=m)
                return cnt + jnp.minimum(rem, 16)

            cnts[bi] = lax.while_loop(pcond, pbody, cnts[bi])

        cp0 = pltpu.async_copy(y0_hbm.at[idx_v.at[pl.ds(SEG[0], KS[0])]],
                               rows_v.at[pl.ds(SEG[0], KS[0])], sem)
        cp1 = pltpu.async_copy(y1_hbm.at[idx_v.at[pl.ds(SEG[1], KS[1])]],
                               rows_v.at[pl.ds(SEG[1], KS[1])], sem)
        cp2 = pltpu.async_copy(y2_hbm.at[idx_v.at[pl.ds(SEG[2], KS[2])]],
                               rows_v.at[pl.ds(SEG[2], KS[2])], sem)
        cp0.wait()
        cp1.wait()
        cp2.wait()

        for bi in range(3):
            seg, k = SEG[bi], KS[bi]

            def red(kk, acc, seg=seg):
                return [jnp.maximum(acc[v], rows_v[seg + kk, pl.ds(v * 16, 16)])
                        for v in range(8)]

            acc = [rows_v[seg, pl.ds(v * 16, 16)] for v in range(8)]
            acc = lax.fori_loop(1, k, red, acc)
            for v in range(8):
                blk_v[c, pl.ds(bi * 128 + v * 16, 16)] = acc[v]
        return 0

    lax.fori_loop(0, _SBLK, per_centroid, 0)
    pltpu.sync_copy(blk_v, pts_hbm.at[pl.ds(b * S + s0, _SBLK), :])


def _sc_call(xf, yf, zf, fps_idx, y0, y1, y2):
    mesh = plsc.VectorSubcoreMesh(core_axis_name="c", subcore_axis_name="s")
    f = pl.kernel(
        _sc_body,
        mesh=mesh,
        compiler_params=pltpu.CompilerParams(needs_layout_passes=False),
        out_type=[
            jax.ShapeDtypeStruct((B * 3 * S,), jnp.float32),
            jax.ShapeDtypeStruct((B * S, 3 * 128), jnp.float32),
        ],
        scratch_types=[
            pltpu.VMEM((N,), jnp.float32),
            pltpu.VMEM((N,), jnp.float32),
            pltpu.VMEM((N,), jnp.float32),
            pltpu.VMEM((_SBLK,), jnp.int32),
            pltpu.VMEM((_SBLK,), jnp.float32),
            pltpu.VMEM((_SBLK,), jnp.float32),
            pltpu.VMEM((_SBLK,), jnp.float32),
            pltpu.VMEM((128,), jnp.int32),
            pltpu.VMEM((TOTK, 128), jnp.float32),
            pltpu.VMEM((_SBLK, 3 * 128), jnp.float32),
            pltpu.SemaphoreType.DMA,
        ],
    )
    return f(xf, yf, zf, fps_idx, y0, y1, y2)


def kernel(xyz, features, W0_0, b0_0, W0_1, b0_1, W1_0, b1_0, W1_1, b1_1,
           W2_0, b2_0, W2_1, b2_1):
    xt = jnp.transpose(xyz, (2, 0, 1))
    fps_idx = _fps(xt)
    feats = features.reshape(B * N, D)
    y0, y1, y2 = _mlp(feats, [W0_0, b0_0, W0_1, b0_1, W1_0, b1_0, W1_1, b1_1,
                              W2_0, b2_0, W2_1, b2_1])
    xf = xyz[:, :, 0].reshape(B * N)
    yf = xyz[:, :, 1].reshape(B * N)
    zf = xyz[:, :, 2].reshape(B * N)
    nx_flat, pts = _sc_call(xf, yf, zf, fps_idx.reshape(B * S), y0, y1, y2)
    new_xyz_t = nx_flat.reshape(B, 3, S)
    return new_xyz_t, jnp.transpose(pts.reshape(B, S, 3 * 128), (0, 2, 1))
